# baseline (speedup 1.0000x reference)
import functools

import jax
import jax.numpy as jnp
from jax import lax
from jax.experimental import pallas as pl
from jax.experimental.pallas import tpu as pltpu
from jax.experimental.pallas import tpu_sc as plsc

E = 64
K = 2
H = 768
I = H * 4
N = 2048
CAP = 128
NA = N * K
SENT = E * CAP

NC = 2
NS = 16
L = 16
NW = NC * NS
APW = NA // NW
TPW = N // NW

I_BLK = 1536
NJ = I // I_BLK
TSH = N // E

_f32 = jnp.float32
_i32 = jnp.int32


def _gelu(v):
    return 0.5 * v * (1.0 + lax.erf(v * 0.7071067811865476))


def _router_body(x_ref, gw_ref, gb_ref, e_ref, w_ref):
    xb = x_ref[...]
    logits = lax.dot_general(xb, gw_ref[...], (((1,), (1,)), ((), ())),
                             preferred_element_type=_f32)
    s = jax.nn.sigmoid(logits + gb_ref[...][None, :])
    ie = lax.broadcasted_iota(_i32, s.shape, 1)
    m0 = jnp.max(s, axis=1, keepdims=True)
    i0 = jnp.min(jnp.where(s == m0, ie, E), axis=1, keepdims=True)
    s2 = jnp.where(ie == i0, -1.0, s)
    m1 = jnp.max(s2, axis=1, keepdims=True)
    i1 = jnp.min(jnp.where(s2 == m1, ie, E), axis=1, keepdims=True)
    denom = m0 + m1 + 1e-6
    e_ref[...] = jnp.concatenate([i0, i1], axis=1)
    w_ref[...] = jnp.concatenate([0.1 * m0 / denom, 0.1 * m1 / denom], axis=1)


def _router(xf, gate_w, gate_bias):
    TB = 2048
    return pl.pallas_call(
        _router_body,
        grid=(N // TB,),
        in_specs=[
            pl.BlockSpec((TB, H), lambda i: (i, 0)),
            pl.BlockSpec((E, H), lambda i: (0, 0)),
            pl.BlockSpec((E,), lambda i: (0,)),
        ],
        out_specs=[
            pl.BlockSpec((TB, K), lambda i: (i, 0)),
            pl.BlockSpec((TB, K), lambda i: (i, 0)),
        ],
        out_shape=[
            jax.ShapeDtypeStruct((N, K), _i32),
            jax.ShapeDtypeStruct((N, K), _f32),
        ],
    )(xf, gate_w, gate_bias)


def _ffn_body(xin_ref, up_ref, upb_ref, dn_ref, dnb_ref,
              xsh_ref, shup_ref, shupb_ref, shdn_ref, shdnb_ref,
              out_ref, sh_ref, acc_ref, accsh_ref):
    j = pl.program_id(1)

    xe = xin_ref[...].astype(jnp.bfloat16)
    hmid = _gelu(jnp.dot(xe, up_ref[0].astype(jnp.bfloat16),
                         preferred_element_type=_f32) + upb_ref[0])
    contrib = jnp.dot(hmid.astype(jnp.bfloat16),
                      dn_ref[0].astype(jnp.bfloat16),
                      preferred_element_type=_f32)

    xb = xsh_ref[...].astype(jnp.bfloat16)
    assert NJ == 2

    def _sh_contrib(jj):
        lo = jj * I_BLK
        hs = _gelu(jnp.dot(xb, shup_ref[:, lo:lo + I_BLK].astype(jnp.bfloat16),
                           preferred_element_type=_f32)
                   + shupb_ref[...][None, lo:lo + I_BLK])
        return jnp.dot(hs.astype(jnp.bfloat16),
                       shdn_ref[lo:lo + I_BLK, :].astype(jnp.bfloat16),
                       preferred_element_type=_f32)

    @pl.when(j == 0)
    def _():
        acc_ref[...] = contrib
        accsh_ref[...] = _sh_contrib(0)

    @pl.when(j == NJ - 1)
    def _():
        out_ref[...] = acc_ref[...] + contrib + dnb_ref[0]
        sh_ref[...] = 0.1 * (accsh_ref[...] + _sh_contrib(NJ - 1)
                             + shdnb_ref[...][None, :])


def _expert_ffn(exp_in, w_up, b_up, w_down, b_down,
                xf, sh_up_w, sh_up_b, sh_down_w, sh_down_b):
    grid = (E, NJ)
    return pl.pallas_call(
        _ffn_body,
        grid=grid,
        in_specs=[
            pl.BlockSpec((CAP, H), lambda e, j: (e, 0)),
            pl.BlockSpec((1, H, I_BLK), lambda e, j: (e, 0, j)),
            pl.BlockSpec((1, 1, I_BLK), lambda e, j: (e * NJ + j, 0, 0)),
            pl.BlockSpec((1, I_BLK, H), lambda e, j: (e, j, 0)),
            pl.BlockSpec((1, 1, H), lambda e, j: (e, 0, 0)),
            pl.BlockSpec((TSH, H), lambda e, j: (e, 0)),
            pl.BlockSpec((H, I), lambda e, j: (0, 0)),
            pl.BlockSpec((I,), lambda e, j: (0,)),
            pl.BlockSpec((I, H), lambda e, j: (0, 0)),
            pl.BlockSpec((H,), lambda e, j: (0,)),
        ],
        out_specs=[
            pl.BlockSpec((CAP, H), lambda e, j: (e, 0)),
            pl.BlockSpec((TSH, H), lambda e, j: (e, 0)),
        ],
        out_shape=[
            jax.ShapeDtypeStruct((E * CAP, H), _f32),
            jax.ShapeDtypeStruct((N, H), _f32),
        ],
        scratch_shapes=[pltpu.VMEM((CAP, H), _f32),
                        pltpu.VMEM((TSH, H), _f32)],
        compiler_params=pltpu.CompilerParams(
            dimension_semantics=("arbitrary", "arbitrary")),
    )(exp_in, w_up, b_up.reshape(E * NJ, 1, I_BLK), w_down,
      b_down.reshape(E, 1, H),
      xf, sh_up_w, sh_up_b, sh_down_w, sh_down_b)


def _occ_within_vec(ev, lane):
    occ = jnp.zeros((L,), _i32)
    for j in range(L - 1):
        sj = jnp.sum(jnp.where(lane == j, ev, 0))
        occ = occ + jnp.where((lane > j) & (ev == sj), 1, 0)
    return occ


def _dispatch_body(eflat_hbm, x_hbm, expin_hbm, slots_hbm,
                   ev_all, counts, slotsl, idx0, idx1, xrows, sem, semx):
    wid = lax.axis_index("s") * NC + lax.axis_index("c")
    base_a = wid * APW
    lane = lax.iota(_i32, L)
    ones = jnp.ones((L,), _i32)

    cx = pltpu.async_copy(x_hbm.at[pl.ds(wid * TPW, TPW)], xrows, semx)
    pltpu.sync_copy(eflat_hbm, ev_all)
    for g in range(E // L):
        counts[pl.ds(g * L, L)] = jnp.zeros((L,), _i32)

    def count_body(i, carry):
        for u in range(4):
            ev = ev_all[pl.ds(i * (4 * L) + u * L, L)]
            plsc.addupdate_scatter(counts, [ev], ones)
        return carry

    lax.fori_loop(0, base_a // (4 * L), count_body, 0)

    for v in range(APW // L):
        ev = ev_all[pl.ds(base_a + v * L, L)]
        base_cnt = plsc.load_gather(counts, [ev])
        pos = base_cnt + _occ_within_vec(ev, lane)
        plsc.addupdate_scatter(counts, [ev], ones)
        slot = ev * CAP + pos
        okm = pos < CAP
        slotsl[pl.ds(v * L, L)] = jnp.where(okm, slot + 1, 0)
        dest = jnp.where(okm, slot, SENT)
        aid = v * L + lane
        lt = lax.shift_right_logical(aid, 1)
        k0 = (aid & 1) == 0
        plsc.store_scatter(idx0, [lt], dest, mask=k0)
        plsc.store_scatter(idx1, [lt], dest, mask=jnp.logical_not(k0))

    pltpu.sync_copy(slotsl, slots_hbm.at[pl.ds(base_a, APW)])
    cx.wait()
    c0 = pltpu.async_copy(xrows, expin_hbm.at[idx0], sem)
    c1 = pltpu.async_copy(xrows, expin_hbm.at[idx1], sem)
    c0.wait()
    c1.wait()


def _dispatch(eflat, xf):
    mesh = plsc.VectorSubcoreMesh(core_axis_name="c", subcore_axis_name="s",
                                  num_cores=NC, num_subcores=NS)
    f = pl.kernel(
        _dispatch_body,
        out_type=(jax.ShapeDtypeStruct((E * CAP + CAP, H), _f32),
                  jax.ShapeDtypeStruct((NA,), _i32)),
        mesh=mesh,
        scratch_types=[
            pltpu.VMEM((NA,), _i32),
            pltpu.VMEM((E,), _i32),
            pltpu.VMEM((APW,), _i32),
            pltpu.VMEM((TPW,), _i32),
            pltpu.VMEM((TPW,), _i32),
            pltpu.VMEM((TPW, H), _f32),
            pltpu.SemaphoreType.DMA,
            pltpu.SemaphoreType.DMA,
        ],
        compiler_params=pltpu.CompilerParams(needs_layout_passes=False),
    )
    return f(eflat, xf)


_TQ = 16
_NQ = TPW // _TQ


def _combine_body(yflat_hbm, sh_hbm, slots_hbm, wflat_hbm, out_hbm,
                  slv, wv, idxv, weffv, ybuf0, ybuf1, shbuf0, shbuf1, outb,
                  semy0, semy1, semsh0, semsh1):
    wid = lax.axis_index("s") * NC + lax.axis_index("c")
    abase = wid * APW
    tbase = wid * TPW
    ybufs, shbufs = (ybuf0, ybuf1), (shbuf0, shbuf1)
    semys, semshs = (semy0, semy1), (semsh0, semsh1)

    pltpu.sync_copy(slots_hbm.at[pl.ds(abase, APW)], slv)
    pltpu.sync_copy(wflat_hbm.at[pl.ds(abase, APW)], wv)
    for v in range(APW // L):
        sl = slv[pl.ds(v * L, L)]
        live = sl > 0
        idxv[pl.ds(v * L, L)] = jnp.where(live, sl - 1, 0)
        weffv[pl.ds(v * L, L)] = jnp.where(live, wv[pl.ds(v * L, L)], 0.0)

    def start(q):
        b = q % 2
        cy = pltpu.async_copy(yflat_hbm.at[idxv.at[pl.ds(q * 2 * _TQ, 2 * _TQ)]],
                              ybufs[b], semys[b])
        cs = pltpu.async_copy(sh_hbm.at[pl.ds(tbase + q * _TQ, _TQ)],
                              shbufs[b], semshs[b])
        return cy, cs

    pend = start(0)
    for q in range(_NQ):
        b = q % 2
        pend[0].wait()
        pend[1].wait()
        if q + 1 < _NQ:
            pend = start(q + 1)
        yrows, shrows = ybufs[b], shbufs[b]

        def tok_body(j, carry):
            w0 = plsc.load_gather(weffv, [jnp.full((L,), 2 * q * _TQ, _i32) + 2 * j])
            w1 = plsc.load_gather(weffv, [jnp.full((L,), 2 * q * _TQ + 1, _i32) + 2 * j])
            acc = []
            mx = jnp.zeros((L,), _f32)
            for c in range(H // L):
                y0 = yrows[2 * j, pl.ds(c * L, L)]
                y1 = yrows[2 * j + 1, pl.ds(c * L, L)]
                shc = shrows[j, pl.ds(c * L, L)]
                a = w0 * y0 + w1 * y1 + shc
                acc.append(a)
                mx = jnp.maximum(mx, jnp.abs(a))
            m16 = jnp.full((L,), jnp.max(mx, axis=0), _f32)
            inv = jnp.ones((L,), _f32) / (m16 + 1e-6)
            for c in range(H // L):
                outb[j, pl.ds(c * L, L)] = acc[c] * inv
            return carry

        lax.fori_loop(0, _TQ, tok_body, 0)
        pltpu.sync_copy(outb, out_hbm.at[pl.ds(tbase + q * _TQ, _TQ)])


def _combine(yflat, sh, slots, wflat):
    mesh = plsc.VectorSubcoreMesh(core_axis_name="c", subcore_axis_name="s",
                                  num_cores=NC, num_subcores=NS)
    f = pl.kernel(
        _combine_body,
        out_type=jax.ShapeDtypeStruct((N, H), _f32),
        mesh=mesh,
        scratch_types=[
            pltpu.VMEM((APW,), _i32),
            pltpu.VMEM((APW,), _f32),
            pltpu.VMEM((APW,), _i32),
            pltpu.VMEM((APW,), _f32),
            pltpu.VMEM((2 * _TQ, H), _f32),
            pltpu.VMEM((2 * _TQ, H), _f32),
            pltpu.VMEM((_TQ, H), _f32),
            pltpu.VMEM((_TQ, H), _f32),
            pltpu.VMEM((_TQ, H), _f32),
            pltpu.SemaphoreType.DMA,
            pltpu.SemaphoreType.DMA,
            pltpu.SemaphoreType.DMA,
            pltpu.SemaphoreType.DMA,
        ],
        compiler_params=pltpu.CompilerParams(needs_layout_passes=False),
    )
    return f(yflat, sh, slots, wflat)


def kernel(x, gate_w, gate_bias, w_up, b_up, w_down, b_down,
           sh_up_w, sh_up_b, sh_down_w, sh_down_b):
    b, s, h = x.shape
    xf = x.reshape(N, H)
    e2, w2 = _router(xf, gate_w, gate_bias)
    eflat = e2.reshape(NA)
    wflat = w2.reshape(NA)
    exp_in, slots = _dispatch(eflat, xf)
    out_flat, sh01 = _expert_ffn(exp_in, w_up, b_up, w_down, b_down,
                                 xf, sh_up_w, sh_up_b, sh_down_w, sh_down_b)
    out = _combine(out_flat, sh01, slots, wflat)
    return out.reshape(b, s, h)


# speedup vs baseline: 1.0031x; 1.0031x over previous
import functools

import jax
import jax.numpy as jnp
from jax import lax
from jax.experimental import pallas as pl
from jax.experimental.pallas import tpu as pltpu
from jax.experimental.pallas import tpu_sc as plsc

E = 64
K = 2
H = 768
I = H * 4
N = 2048
CAP = 128
NA = N * K
SENT = E * CAP

NC = 2
NS = 16
L = 16
NW = NC * NS
APW = NA // NW
TPW = N // NW

I_BLK = 1536
NJ = I // I_BLK
TSH = N // E

_f32 = jnp.float32
_i32 = jnp.int32


def _gelu(v):
    return 0.5 * v * (1.0 + lax.erf(v * 0.7071067811865476))


def _router_body(x_ref, gw_ref, gb_ref, e_ref, w_ref):
    xb = x_ref[...]
    logits = lax.dot_general(xb, gw_ref[...], (((1,), (1,)), ((), ())),
                             preferred_element_type=_f32)
    s = jax.nn.sigmoid(logits + gb_ref[...][None, :])
    ie = lax.broadcasted_iota(_i32, s.shape, 1)
    m0 = jnp.max(s, axis=1, keepdims=True)
    i0 = jnp.min(jnp.where(s == m0, ie, E), axis=1, keepdims=True)
    s2 = jnp.where(ie == i0, -1.0, s)
    m1 = jnp.max(s2, axis=1, keepdims=True)
    i1 = jnp.min(jnp.where(s2 == m1, ie, E), axis=1, keepdims=True)
    denom = m0 + m1 + 1e-6
    e_ref[...] = jnp.concatenate([i0, i1], axis=1)
    w_ref[...] = jnp.concatenate([0.1 * m0 / denom, 0.1 * m1 / denom], axis=1)


def _router(xf, gate_w, gate_bias):
    TB = 1024
    return pl.pallas_call(
        _router_body,
        grid=(N // TB,),
        in_specs=[
            pl.BlockSpec((TB, H), lambda i: (i, 0)),
            pl.BlockSpec((E, H), lambda i: (0, 0)),
            pl.BlockSpec((E,), lambda i: (0,)),
        ],
        out_specs=[
            pl.BlockSpec((TB, K), lambda i: (i, 0)),
            pl.BlockSpec((TB, K), lambda i: (i, 0)),
        ],
        out_shape=[
            jax.ShapeDtypeStruct((N, K), _i32),
            jax.ShapeDtypeStruct((N, K), _f32),
        ],
    )(xf, gate_w, gate_bias)


def _ffn_body(xin_ref, up_ref, upb_ref, dn_ref, dnb_ref,
              xsh_ref, shup_ref, shupb_ref, shdn_ref, shdnb_ref,
              out_ref, sh_ref, acc_ref, accsh_ref):
    j = pl.program_id(1)

    xe = xin_ref[...].astype(jnp.bfloat16)
    hmid = _gelu(jnp.dot(xe, up_ref[0].astype(jnp.bfloat16),
                         preferred_element_type=_f32) + upb_ref[0])
    contrib = jnp.dot(hmid.astype(jnp.bfloat16),
                      dn_ref[0].astype(jnp.bfloat16),
                      preferred_element_type=_f32)

    xb = xsh_ref[...].astype(jnp.bfloat16)
    assert NJ == 2

    def _sh_contrib(jj):
        lo = jj * I_BLK
        hs = _gelu(jnp.dot(xb, shup_ref[:, lo:lo + I_BLK].astype(jnp.bfloat16),
                           preferred_element_type=_f32)
                   + shupb_ref[...][None, lo:lo + I_BLK])
        return jnp.dot(hs.astype(jnp.bfloat16),
                       shdn_ref[lo:lo + I_BLK, :].astype(jnp.bfloat16),
                       preferred_element_type=_f32)

    @pl.when(j == 0)
    def _():
        acc_ref[...] = contrib
        accsh_ref[...] = _sh_contrib(0)

    @pl.when(j == NJ - 1)
    def _():
        out_ref[...] = acc_ref[...] + contrib + dnb_ref[0]
        sh_ref[...] = 0.1 * (accsh_ref[...] + _sh_contrib(NJ - 1)
                             + shdnb_ref[...][None, :])


def _expert_ffn(exp_in, w_up, b_up, w_down, b_down,
                xf, sh_up_w, sh_up_b, sh_down_w, sh_down_b):
    grid = (E, NJ)
    return pl.pallas_call(
        _ffn_body,
        grid=grid,
        in_specs=[
            pl.BlockSpec((CAP, H), lambda e, j: (e, 0)),
            pl.BlockSpec((1, H, I_BLK), lambda e, j: (e, 0, j)),
            pl.BlockSpec((1, 1, I_BLK), lambda e, j: (e * NJ + j, 0, 0)),
            pl.BlockSpec((1, I_BLK, H), lambda e, j: (e, j, 0)),
            pl.BlockSpec((1, 1, H), lambda e, j: (e, 0, 0)),
            pl.BlockSpec((TSH, H), lambda e, j: (e, 0)),
            pl.BlockSpec((H, I), lambda e, j: (0, 0)),
            pl.BlockSpec((I,), lambda e, j: (0,)),
            pl.BlockSpec((I, H), lambda e, j: (0, 0)),
            pl.BlockSpec((H,), lambda e, j: (0,)),
        ],
        out_specs=[
            pl.BlockSpec((CAP, H), lambda e, j: (e, 0)),
            pl.BlockSpec((TSH, H), lambda e, j: (e, 0)),
        ],
        out_shape=[
            jax.ShapeDtypeStruct((E * CAP, H), _f32),
            jax.ShapeDtypeStruct((N, H), _f32),
        ],
        scratch_shapes=[pltpu.VMEM((CAP, H), _f32),
                        pltpu.VMEM((TSH, H), _f32)],
        compiler_params=pltpu.CompilerParams(
            dimension_semantics=("arbitrary", "arbitrary")),
    )(exp_in, w_up, b_up.reshape(E * NJ, 1, I_BLK), w_down,
      b_down.reshape(E, 1, H),
      xf, sh_up_w, sh_up_b, sh_down_w, sh_down_b)


def _occ_within_vec(ev, lane):
    occ = jnp.zeros((L,), _i32)
    for j in range(L - 1):
        sj = jnp.sum(jnp.where(lane == j, ev, 0))
        occ = occ + jnp.where((lane > j) & (ev == sj), 1, 0)
    return occ


def _dispatch_body(eflat_hbm, x_hbm, expin_hbm, slots_hbm,
                   ev_all, counts, slotsl, idx0, idx1, xrows, sem, semx):
    wid = lax.axis_index("s") * NC + lax.axis_index("c")
    base_a = wid * APW
    lane = lax.iota(_i32, L)
    ones = jnp.ones((L,), _i32)

    cx = pltpu.async_copy(x_hbm.at[pl.ds(wid * TPW, TPW)], xrows, semx)
    pltpu.sync_copy(eflat_hbm, ev_all)
    for g in range(E // L):
        counts[pl.ds(g * L, L)] = jnp.zeros((L,), _i32)

    def count_body(i, carry):
        for u in range(4):
            ev = ev_all[pl.ds(i * (4 * L) + u * L, L)]
            plsc.addupdate_scatter(counts, [ev], ones)
        return carry

    lax.fori_loop(0, base_a // (4 * L), count_body, 0)

    for v in range(APW // L):
        ev = ev_all[pl.ds(base_a + v * L, L)]
        base_cnt = plsc.load_gather(counts, [ev])
        pos = base_cnt + _occ_within_vec(ev, lane)
        plsc.addupdate_scatter(counts, [ev], ones)
        slot = ev * CAP + pos
        okm = pos < CAP
        slotsl[pl.ds(v * L, L)] = jnp.where(okm, slot + 1, 0)
        dest = jnp.where(okm, slot, SENT)
        aid = v * L + lane
        lt = lax.shift_right_logical(aid, 1)
        k0 = (aid & 1) == 0
        plsc.store_scatter(idx0, [lt], dest, mask=k0)
        plsc.store_scatter(idx1, [lt], dest, mask=jnp.logical_not(k0))

    pltpu.sync_copy(slotsl, slots_hbm.at[pl.ds(base_a, APW)])
    cx.wait()
    c0 = pltpu.async_copy(xrows, expin_hbm.at[idx0], sem)
    c1 = pltpu.async_copy(xrows, expin_hbm.at[idx1], sem)
    c0.wait()
    c1.wait()


def _dispatch(eflat, xf):
    mesh = plsc.VectorSubcoreMesh(core_axis_name="c", subcore_axis_name="s",
                                  num_cores=NC, num_subcores=NS)
    f = pl.kernel(
        _dispatch_body,
        out_type=(jax.ShapeDtypeStruct((E * CAP + CAP, H), _f32),
                  jax.ShapeDtypeStruct((NA,), _i32)),
        mesh=mesh,
        scratch_types=[
            pltpu.VMEM((NA,), _i32),
            pltpu.VMEM((E,), _i32),
            pltpu.VMEM((APW,), _i32),
            pltpu.VMEM((TPW,), _i32),
            pltpu.VMEM((TPW,), _i32),
            pltpu.VMEM((TPW, H), _f32),
            pltpu.SemaphoreType.DMA,
            pltpu.SemaphoreType.DMA,
        ],
        compiler_params=pltpu.CompilerParams(needs_layout_passes=False),
    )
    return f(eflat, xf)


_TQ = 16
_NQ = TPW // _TQ


def _combine_body(yflat_hbm, sh_hbm, slots_hbm, wflat_hbm, out_hbm,
                  slv, wv, idxv, weffv, ybuf0, ybuf1, shbuf0, shbuf1, outb,
                  semy0, semy1, semsh0, semsh1):
    wid = lax.axis_index("s") * NC + lax.axis_index("c")
    abase = wid * APW
    tbase = wid * TPW
    ybufs, shbufs = (ybuf0, ybuf1), (shbuf0, shbuf1)
    semys, semshs = (semy0, semy1), (semsh0, semsh1)

    pltpu.sync_copy(slots_hbm.at[pl.ds(abase, APW)], slv)
    pltpu.sync_copy(wflat_hbm.at[pl.ds(abase, APW)], wv)
    for v in range(APW // L):
        sl = slv[pl.ds(v * L, L)]
        live = sl > 0
        idxv[pl.ds(v * L, L)] = jnp.where(live, sl - 1, 0)
        weffv[pl.ds(v * L, L)] = jnp.where(live, wv[pl.ds(v * L, L)], 0.0)

    def start(q):
        b = q % 2
        cy = pltpu.async_copy(yflat_hbm.at[idxv.at[pl.ds(q * 2 * _TQ, 2 * _TQ)]],
                              ybufs[b], semys[b])
        cs = pltpu.async_copy(sh_hbm.at[pl.ds(tbase + q * _TQ, _TQ)],
                              shbufs[b], semshs[b])
        return cy, cs

    pend = start(0)
    for q in range(_NQ):
        b = q % 2
        pend[0].wait()
        pend[1].wait()
        if q + 1 < _NQ:
            pend = start(q + 1)
        yrows, shrows = ybufs[b], shbufs[b]

        def tok_body(j, carry):
            w0 = plsc.load_gather(weffv, [jnp.full((L,), 2 * q * _TQ, _i32) + 2 * j])
            w1 = plsc.load_gather(weffv, [jnp.full((L,), 2 * q * _TQ + 1, _i32) + 2 * j])
            acc = []
            mx = jnp.zeros((L,), _f32)
            for c in range(H // L):
                y0 = yrows[2 * j, pl.ds(c * L, L)]
                y1 = yrows[2 * j + 1, pl.ds(c * L, L)]
                shc = shrows[j, pl.ds(c * L, L)]
                a = w0 * y0 + w1 * y1 + shc
                acc.append(a)
                mx = jnp.maximum(mx, jnp.abs(a))
            m16 = jnp.full((L,), jnp.max(mx, axis=0), _f32)
            inv = jnp.ones((L,), _f32) / (m16 + 1e-6)
            for c in range(H // L):
                outb[j, pl.ds(c * L, L)] = acc[c] * inv
            return carry

        lax.fori_loop(0, _TQ, tok_body, 0)
        pltpu.sync_copy(outb, out_hbm.at[pl.ds(tbase + q * _TQ, _TQ)])


def _combine(yflat, sh, slots, wflat):
    mesh = plsc.VectorSubcoreMesh(core_axis_name="c", subcore_axis_name="s",
                                  num_cores=NC, num_subcores=NS)
    f = pl.kernel(
        _combine_body,
        out_type=jax.ShapeDtypeStruct((N, H), _f32),
        mesh=mesh,
        scratch_types=[
            pltpu.VMEM((APW,), _i32),
            pltpu.VMEM((APW,), _f32),
            pltpu.VMEM((APW,), _i32),
            pltpu.VMEM((APW,), _f32),
            pltpu.VMEM((2 * _TQ, H), _f32),
            pltpu.VMEM((2 * _TQ, H), _f32),
            pltpu.VMEM((_TQ, H), _f32),
            pltpu.VMEM((_TQ, H), _f32),
            pltpu.VMEM((_TQ, H), _f32),
            pltpu.SemaphoreType.DMA,
            pltpu.SemaphoreType.DMA,
            pltpu.SemaphoreType.DMA,
            pltpu.SemaphoreType.DMA,
        ],
        compiler_params=pltpu.CompilerParams(needs_layout_passes=False),
    )
    return f(yflat, sh, slots, wflat)


def kernel(x, gate_w, gate_bias, w_up, b_up, w_down, b_down,
           sh_up_w, sh_up_b, sh_down_w, sh_down_b):
    b, s, h = x.shape
    xf = x.reshape(N, H)
    e2, w2 = _router(xf, gate_w, gate_bias)
    eflat = e2.reshape(NA)
    wflat = w2.reshape(NA)
    exp_in, slots = _dispatch(eflat, xf)
    out_flat, sh01 = _expert_ffn(exp_in, w_up, b_up, w_down, b_down,
                                 xf, sh_up_w, sh_up_b, sh_down_w, sh_down_b)
    out = _combine(out_flat, sh01, slots, wflat)
    return out.reshape(b, s, h)


# speedup vs baseline: 1.0037x; 1.0006x over previous
import jax
import jax.numpy as jnp
from jax import lax
from jax.experimental import pallas as pl
from jax.experimental.pallas import tpu as pltpu
from jax.experimental.pallas import tpu_sc as plsc

E = 64
K = 2
H = 768
I = H * 4
N = 2048
CAP = 128
NA = N * K
SENT = E * CAP

NC = 2
NS = 16
L = 16
NW = NC * NS
APW = NA // NW
TPW = N // NW

I_BLK = 1536
NJ = I // I_BLK
TSH = N // E

_f32 = jnp.float32
_i32 = jnp.int32


def _gelu(v):
    return 0.5 * v * (1.0 + lax.erf(v * 0.7071067811865476))


def _router_body(x_ref, gw_ref, gb_ref, e_ref, w_ref):
    xb = x_ref[...]
    logits = lax.dot_general(xb, gw_ref[...], (((1,), (1,)), ((), ())),
                             preferred_element_type=_f32)
    s = jax.nn.sigmoid(logits + gb_ref[...][None, :])
    ie = lax.broadcasted_iota(_i32, s.shape, 1)
    m0 = jnp.max(s, axis=1, keepdims=True)
    i0 = jnp.min(jnp.where(s == m0, ie, E), axis=1, keepdims=True)
    s2 = jnp.where(ie == i0, -1.0, s)
    m1 = jnp.max(s2, axis=1, keepdims=True)
    i1 = jnp.min(jnp.where(s2 == m1, ie, E), axis=1, keepdims=True)
    denom = m0 + m1 + 1e-6
    e_ref[...] = jnp.concatenate([i0, i1], axis=1)
    w_ref[...] = jnp.concatenate([0.1 * m0 / denom, 0.1 * m1 / denom], axis=1)


def _router(xf, gate_w, gate_bias):
    TB = 1024
    return pl.pallas_call(
        _router_body,
        grid=(N // TB,),
        in_specs=[
            pl.BlockSpec((TB, H), lambda i: (i, 0)),
            pl.BlockSpec((E, H), lambda i: (0, 0)),
            pl.BlockSpec((E,), lambda i: (0,)),
        ],
        out_specs=[
            pl.BlockSpec((TB, K), lambda i: (i, 0)),
            pl.BlockSpec((TB, K), lambda i: (i, 0)),
        ],
        out_shape=[
            jax.ShapeDtypeStruct((N, K), _i32),
            jax.ShapeDtypeStruct((N, K), _f32),
        ],
    )(xf, gate_w, gate_bias)


def _ffn_body(xin_ref, up_ref, upb_ref, dn_ref, dnb_ref,
              xsh_ref, shup_ref, shupb_ref, shdn_ref, shdnb_ref,
              out_ref, sh_ref, acc_ref, accsh_ref):
    j = pl.program_id(1)

    xe = xin_ref[...].astype(jnp.bfloat16)
    hmid = _gelu(jnp.dot(xe, up_ref[0].astype(jnp.bfloat16),
                         preferred_element_type=_f32) + upb_ref[0])
    contrib = jnp.dot(hmid.astype(jnp.bfloat16),
                      dn_ref[0].astype(jnp.bfloat16),
                      preferred_element_type=_f32)

    xb = xsh_ref[...].astype(jnp.bfloat16)
    assert NJ == 2

    def _sh_contrib(jj):
        lo = jj * I_BLK
        hs = _gelu(jnp.dot(xb, shup_ref[:, lo:lo + I_BLK].astype(jnp.bfloat16),
                           preferred_element_type=_f32)
                   + shupb_ref[...][None, lo:lo + I_BLK])
        return jnp.dot(hs.astype(jnp.bfloat16),
                       shdn_ref[lo:lo + I_BLK, :].astype(jnp.bfloat16),
                       preferred_element_type=_f32)

    @pl.when(j == 0)
    def _():
        acc_ref[...] = contrib
        accsh_ref[...] = _sh_contrib(0)

    @pl.when(j == NJ - 1)
    def _():
        out_ref[...] = acc_ref[...] + contrib + dnb_ref[0]
        sh_ref[...] = 0.1 * (accsh_ref[...] + _sh_contrib(NJ - 1)
                             + shdnb_ref[...][None, :])


def _expert_ffn(exp_in, w_up, b_up, w_down, b_down,
                xf, sh_up_w, sh_up_b, sh_down_w, sh_down_b):
    grid = (E, NJ)
    return pl.pallas_call(
        _ffn_body,
        grid=grid,
        in_specs=[
            pl.BlockSpec((CAP, H), lambda e, j: (e, 0)),
            pl.BlockSpec((1, H, I_BLK), lambda e, j: (e, 0, j)),
            pl.BlockSpec((1, 1, I_BLK), lambda e, j: (e * NJ + j, 0, 0)),
            pl.BlockSpec((1, I_BLK, H), lambda e, j: (e, j, 0)),
            pl.BlockSpec((1, 1, H), lambda e, j: (e, 0, 0)),
            pl.BlockSpec((TSH, H), lambda e, j: (e, 0)),
            pl.BlockSpec((H, I), lambda e, j: (0, 0)),
            pl.BlockSpec((I,), lambda e, j: (0,)),
            pl.BlockSpec((I, H), lambda e, j: (0, 0)),
            pl.BlockSpec((H,), lambda e, j: (0,)),
        ],
        out_specs=[
            pl.BlockSpec((CAP, H), lambda e, j: (e, 0)),
            pl.BlockSpec((TSH, H), lambda e, j: (e, 0)),
        ],
        out_shape=[
            jax.ShapeDtypeStruct((E * CAP, H), _f32),
            jax.ShapeDtypeStruct((N, H), _f32),
        ],
        scratch_shapes=[pltpu.VMEM((CAP, H), _f32),
                        pltpu.VMEM((TSH, H), _f32)],
        compiler_params=pltpu.CompilerParams(
            dimension_semantics=("arbitrary", "arbitrary")),
    )(exp_in, w_up, b_up.reshape(E * NJ, 1, I_BLK), w_down,
      b_down.reshape(E, 1, H),
      xf, sh_up_w, sh_up_b, sh_down_w, sh_down_b)


def _occ_within_vec(ev, lane):
    occ = jnp.zeros((L,), _i32)
    for j in range(L - 1):
        sj = jnp.sum(jnp.where(lane == j, ev, 0))
        occ = occ + jnp.where((lane > j) & (ev == sj), 1, 0)
    return occ


def _dispatch_body(eflat_hbm, x_hbm, expin_hbm, slots_hbm,
                   ev_all, counts, slotsl, idx0, idx1, xrows, sem, semx):
    wid = lax.axis_index("s") * NC + lax.axis_index("c")
    base_a = wid * APW
    lane = lax.iota(_i32, L)
    ones = jnp.ones((L,), _i32)

    cx = pltpu.async_copy(x_hbm.at[pl.ds(wid * TPW, TPW)], xrows, semx)
    pltpu.sync_copy(eflat_hbm, ev_all)
    for g in range(E // L):
        counts[pl.ds(g * L, L)] = jnp.zeros((L,), _i32)

    def count_body(i, carry):
        for u in range(4):
            ev = ev_all[pl.ds(i * (4 * L) + u * L, L)]
            plsc.addupdate_scatter(counts, [ev], ones)
        return carry

    lax.fori_loop(0, base_a // (4 * L), count_body, 0)

    for v in range(APW // L):
        ev = ev_all[pl.ds(base_a + v * L, L)]
        base_cnt = plsc.load_gather(counts, [ev])
        pos = base_cnt + _occ_within_vec(ev, lane)
        plsc.addupdate_scatter(counts, [ev], ones)
        slot = ev * CAP + pos
        okm = pos < CAP
        slotsl[pl.ds(v * L, L)] = jnp.where(okm, slot + 1, 0)
        dest = jnp.where(okm, slot, SENT)
        aid = v * L + lane
        lt = lax.shift_right_logical(aid, 1)
        k0 = (aid & 1) == 0
        plsc.store_scatter(idx0, [lt], dest, mask=k0)
        plsc.store_scatter(idx1, [lt], dest, mask=jnp.logical_not(k0))

    pltpu.sync_copy(slotsl, slots_hbm.at[pl.ds(base_a, APW)])
    cx.wait()
    c0 = pltpu.async_copy(xrows, expin_hbm.at[idx0], sem)
    c1 = pltpu.async_copy(xrows, expin_hbm.at[idx1], sem)
    c0.wait()
    c1.wait()


def _dispatch(eflat, xf):
    mesh = plsc.VectorSubcoreMesh(core_axis_name="c", subcore_axis_name="s",
                                  num_cores=NC, num_subcores=NS)
    f = pl.kernel(
        _dispatch_body,
        out_type=(jax.ShapeDtypeStruct((E * CAP + CAP, H), _f32),
                  jax.ShapeDtypeStruct((NA,), _i32)),
        mesh=mesh,
        scratch_types=[
            pltpu.VMEM((NA,), _i32),
            pltpu.VMEM((E,), _i32),
            pltpu.VMEM((APW,), _i32),
            pltpu.VMEM((TPW,), _i32),
            pltpu.VMEM((TPW,), _i32),
            pltpu.VMEM((TPW, H), _f32),
            pltpu.SemaphoreType.DMA,
            pltpu.SemaphoreType.DMA,
        ],
        compiler_params=pltpu.CompilerParams(needs_layout_passes=False),
    )
    return f(eflat, xf)


_TQ = 16
_NQ = TPW // _TQ


def _combine_body(yflat_hbm, sh_hbm, slots_hbm, wflat_hbm, out_hbm,
                  slv, wv, idxv, weffv, ybuf0, ybuf1, shbuf0, shbuf1, outb,
                  semy0, semy1, semsh0, semsh1):
    wid = lax.axis_index("s") * NC + lax.axis_index("c")
    abase = wid * APW
    tbase = wid * TPW
    ybufs, shbufs = (ybuf0, ybuf1), (shbuf0, shbuf1)
    semys, semshs = (semy0, semy1), (semsh0, semsh1)

    pltpu.sync_copy(slots_hbm.at[pl.ds(abase, APW)], slv)
    pltpu.sync_copy(wflat_hbm.at[pl.ds(abase, APW)], wv)
    for v in range(APW // L):
        sl = slv[pl.ds(v * L, L)]
        live = sl > 0
        idxv[pl.ds(v * L, L)] = jnp.where(live, sl - 1, 0)
        weffv[pl.ds(v * L, L)] = jnp.where(live, wv[pl.ds(v * L, L)], 0.0)

    def start(q):
        b = q % 2
        cy = pltpu.async_copy(yflat_hbm.at[idxv.at[pl.ds(q * 2 * _TQ, 2 * _TQ)]],
                              ybufs[b], semys[b])
        cs = pltpu.async_copy(sh_hbm.at[pl.ds(tbase + q * _TQ, _TQ)],
                              shbufs[b], semshs[b])
        return cy, cs

    pend = start(0)
    for q in range(_NQ):
        b = q % 2
        pend[0].wait()
        pend[1].wait()
        if q + 1 < _NQ:
            pend = start(q + 1)
        yrows, shrows = ybufs[b], shbufs[b]

        def tok_body(j, carry):
            w0 = plsc.load_gather(weffv, [jnp.full((L,), 2 * q * _TQ, _i32) + 2 * j])
            w1 = plsc.load_gather(weffv, [jnp.full((L,), 2 * q * _TQ + 1, _i32) + 2 * j])
            acc = []
            mx = jnp.zeros((L,), _f32)
            for c in range(H // L):
                y0 = yrows[2 * j, pl.ds(c * L, L)]
                y1 = yrows[2 * j + 1, pl.ds(c * L, L)]
                shc = shrows[j, pl.ds(c * L, L)]
                a = w0 * y0 + w1 * y1 + shc
                acc.append(a)
                mx = jnp.maximum(mx, jnp.abs(a))
            m16 = jnp.full((L,), jnp.max(mx, axis=0), _f32)
            inv = jnp.ones((L,), _f32) / (m16 + 1e-6)
            for c in range(H // L):
                outb[j, pl.ds(c * L, L)] = acc[c] * inv
            return carry

        lax.fori_loop(0, _TQ, tok_body, 0)
        pltpu.sync_copy(outb, out_hbm.at[pl.ds(tbase + q * _TQ, _TQ)])


def _combine(yflat, sh, slots, wflat):
    mesh = plsc.VectorSubcoreMesh(core_axis_name="c", subcore_axis_name="s",
                                  num_cores=NC, num_subcores=NS)
    f = pl.kernel(
        _combine_body,
        out_type=jax.ShapeDtypeStruct((N, H), _f32),
        mesh=mesh,
        scratch_types=[
            pltpu.VMEM((APW,), _i32),
            pltpu.VMEM((APW,), _f32),
            pltpu.VMEM((APW,), _i32),
            pltpu.VMEM((APW,), _f32),
            pltpu.VMEM((2 * _TQ, H), _f32),
            pltpu.VMEM((2 * _TQ, H), _f32),
            pltpu.VMEM((_TQ, H), _f32),
            pltpu.VMEM((_TQ, H), _f32),
            pltpu.VMEM((_TQ, H), _f32),
            pltpu.SemaphoreType.DMA,
            pltpu.SemaphoreType.DMA,
            pltpu.SemaphoreType.DMA,
            pltpu.SemaphoreType.DMA,
        ],
        compiler_params=pltpu.CompilerParams(needs_layout_passes=False),
    )
    return f(yflat, sh, slots, wflat)


def kernel(x, gate_w, gate_bias, w_up, b_up, w_down, b_down,
           sh_up_w, sh_up_b, sh_down_w, sh_down_b):
    b, s, h = x.shape
    xf = x.reshape(N, H)
    e2, w2 = _router(xf, gate_w, gate_bias)
    eflat = e2.reshape(NA)
    wflat = w2.reshape(NA)
    exp_in, slots = _dispatch(eflat, xf)
    out_flat, sh01 = _expert_ffn(exp_in, w_up, b_up, w_down, b_down,
                                 xf, sh_up_w, sh_up_b, sh_down_w, sh_down_b)
    out = _combine(out_flat, sh01, slots, wflat)
    return out.reshape(b, s, h)
